# baseline (speedup 1.0000x reference)
import functools

import jax
import jax.numpy as jnp
from jax import lax
from jax.experimental import pallas as pl
from jax.experimental.pallas import tpu as pltpu
from jax.experimental.pallas import tpu_sc as plsc


def _make_lookup(vocab, embed, batch):
    info = plsc.get_sparse_core_info()
    num_cores, num_subcores = info.num_cores, info.num_subcores
    num_workers = num_cores * num_subcores
    assert batch % num_workers == 0
    n = batch // num_workers

    mesh = plsc.VectorSubcoreMesh(core_axis_name="c", subcore_axis_name="s")

    @functools.partial(
        pl.kernel,
        mesh=mesh,
        compiler_params=pltpu.CompilerParams(use_tc_tiling_on_sc=False),
        out_type=jax.ShapeDtypeStruct((2, batch, embed), jnp.float32),
        scratch_types=[
            pltpu.VMEM((n,), jnp.int32),
            pltpu.VMEM((n,), jnp.int32),
            pltpu.VMEM((n, embed), jnp.float32),
            pltpu.VMEM((n, embed), jnp.float32),
            pltpu.SemaphoreType.DMA,
            pltpu.SemaphoreType.DMA,
        ],
    )
    def lookup(user_hbm, item_hbm, table_hbm, out_hbm,
               idx_u, idx_i, rows_u, rows_i, sem_u, sem_i):
        wid = lax.axis_index("s") * num_cores + lax.axis_index("c")
        base = wid * n
        pltpu.sync_copy(user_hbm.at[pl.ds(base, n)], idx_u)
        pltpu.sync_copy(item_hbm.at[pl.ds(base, n)], idx_i)
        cp_u = pltpu.async_copy(table_hbm.at[idx_u], rows_u, sem_u)
        cp_i = pltpu.async_copy(table_hbm.at[idx_i], rows_i, sem_i)
        cp_u.wait()
        pltpu.sync_copy(rows_u, out_hbm.at[0, pl.ds(base, n), :])
        cp_i.wait()
        pltpu.sync_copy(rows_i, out_hbm.at[1, pl.ds(base, n), :])

    return lookup


def kernel(user, item, hash_embeds_weight):
    vocab, embed = hash_embeds_weight.shape
    (batch,) = user.shape
    lookup = _make_lookup(vocab, embed, batch)
    pair = lookup(user, item, hash_embeds_weight)
    return jnp.concatenate([pair[0], pair[1]], axis=1)


# speedup vs baseline: 1.0082x; 1.0082x over previous
import functools

import jax
import jax.numpy as jnp
from jax import lax
from jax.experimental import pallas as pl
from jax.experimental.pallas import tpu as pltpu
from jax.experimental.pallas import tpu_sc as plsc


def _make_lookup(vocab, embed, batch):
    info = plsc.get_sparse_core_info()
    num_cores, num_subcores = info.num_cores, info.num_subcores
    num_workers = num_cores * num_subcores
    assert batch % num_workers == 0
    n = batch // num_workers

    mesh = plsc.VectorSubcoreMesh(core_axis_name="c", subcore_axis_name="s")

    @functools.partial(
        pl.kernel,
        mesh=mesh,
        compiler_params=pltpu.CompilerParams(use_tc_tiling_on_sc=False),
        out_type=(
            jax.ShapeDtypeStruct((batch, embed), jnp.float32),
            jax.ShapeDtypeStruct((batch, embed), jnp.float32),
        ),
        scratch_types=[
            pltpu.VMEM((n,), jnp.int32),
            pltpu.VMEM((n,), jnp.int32),
            pltpu.VMEM((n, embed), jnp.float32),
            pltpu.VMEM((n, embed), jnp.float32),
            pltpu.SemaphoreType.DMA,
            pltpu.SemaphoreType.DMA,
        ],
    )
    def lookup(user_hbm, item_hbm, table_hbm, out_u_hbm, out_i_hbm,
               idx_u, idx_i, rows_u, rows_i, sem_u, sem_i):
        wid = lax.axis_index("s") * num_cores + lax.axis_index("c")
        base = wid * n
        pltpu.sync_copy(user_hbm.at[pl.ds(base, n)], idx_u)
        pltpu.sync_copy(item_hbm.at[pl.ds(base, n)], idx_i)
        cp_u = pltpu.async_copy(table_hbm.at[idx_u], rows_u, sem_u)
        cp_i = pltpu.async_copy(table_hbm.at[idx_i], rows_i, sem_i)
        cp_u.wait()
        pltpu.sync_copy(rows_u, out_u_hbm.at[pl.ds(base, n)])
        cp_i.wait()
        pltpu.sync_copy(rows_i, out_i_hbm.at[pl.ds(base, n)])

    return lookup


def kernel(user, item, hash_embeds_weight):
    vocab, embed = hash_embeds_weight.shape
    (batch,) = user.shape
    lookup = _make_lookup(vocab, embed, batch)
    out_u, out_i = lookup(user, item, hash_embeds_weight)
    return jnp.concatenate([out_u, out_i], axis=1)


# speedup vs baseline: 1.0371x; 1.0287x over previous
import functools

import jax
import jax.numpy as jnp
from jax import lax
from jax.experimental import pallas as pl
from jax.experimental.pallas import tpu as pltpu
from jax.experimental.pallas import tpu_sc as plsc


def _make_lookup(embed, total):
    info = plsc.get_sparse_core_info()
    num_cores, num_subcores = info.num_cores, info.num_subcores
    num_workers = num_cores * num_subcores
    assert total % num_workers == 0
    n = total // num_workers

    mesh = plsc.VectorSubcoreMesh(core_axis_name="c", subcore_axis_name="s")

    @functools.partial(
        pl.kernel,
        mesh=mesh,
        compiler_params=pltpu.CompilerParams(use_tc_tiling_on_sc=False),
        out_type=jax.ShapeDtypeStruct((total, embed), jnp.float32),
        scratch_types=[
            pltpu.VMEM((n,), jnp.int32),
            pltpu.VMEM((n, embed), jnp.float32),
            pltpu.SemaphoreType.DMA,
        ],
    )
    def lookup(idx_hbm, table_hbm, out_hbm, idx, rows, sem):
        wid = lax.axis_index("s") * num_cores + lax.axis_index("c")
        base = wid * n
        pltpu.sync_copy(idx_hbm.at[pl.ds(base, n)], idx)
        cp = pltpu.async_copy(table_hbm.at[idx], rows, sem)
        cp.wait()
        pltpu.sync_copy(rows, out_hbm.at[pl.ds(base, n)])

    return lookup


def kernel(user, item, hash_embeds_weight):
    vocab, embed = hash_embeds_weight.shape
    (batch,) = user.shape
    idx = jnp.stack([user, item], axis=1).reshape(-1)
    lookup = _make_lookup(embed, 2 * batch)
    out = lookup(idx, hash_embeds_weight)
    return out.reshape(batch, 2 * embed)


# speedup vs baseline: 1.1151x; 1.0752x over previous
import functools

import jax
import jax.numpy as jnp
from jax import lax
from jax.experimental import pallas as pl
from jax.experimental.pallas import tpu as pltpu
from jax.experimental.pallas import tpu_sc as plsc


def _make_lookup(vocab, total):
    info = plsc.get_sparse_core_info()
    num_cores, num_subcores = info.num_cores, info.num_subcores
    num_workers = num_cores * num_subcores
    assert total % num_workers == 0
    n = total // num_workers
    ch = min(n, 256)
    assert n % ch == 0 and (n // ch) % 2 == 0

    mesh = plsc.VectorSubcoreMesh(core_axis_name="c", subcore_axis_name="s")

    @functools.partial(
        pl.kernel,
        mesh=mesh,
        compiler_params=pltpu.CompilerParams(use_tc_tiling_on_sc=True),
        out_type=jax.ShapeDtypeStruct((total, 128), jnp.float32),
        scratch_types=[
            pltpu.VMEM((n,), jnp.int32),
            pltpu.VMEM((ch, 128), jnp.float32),
            pltpu.VMEM((ch, 128), jnp.float32),
            pltpu.SemaphoreType.DMA,
            pltpu.SemaphoreType.DMA,
        ],
    )
    def lookup(idx_hbm, table_hbm, out_hbm, idx, rows0, rows1, sem0, sem1):
        wid = lax.axis_index("s") * num_cores + lax.axis_index("c")
        base = wid * n
        pltpu.sync_copy(idx_hbm.at[pl.ds(base, n)], idx)
        for c in range(0, n // ch, 2):
            cp0 = pltpu.async_copy(
                table_hbm.at[idx.at[pl.ds(c * ch, ch)]], rows0, sem0)
            cp1 = pltpu.async_copy(
                table_hbm.at[idx.at[pl.ds((c + 1) * ch, ch)]], rows1, sem1)
            cp0.wait()
            pltpu.sync_copy(rows0, out_hbm.at[pl.ds(base + c * ch, ch)])
            cp1.wait()
            pltpu.sync_copy(rows1, out_hbm.at[pl.ds(base + (c + 1) * ch, ch)])

    return lookup


def kernel(user, item, hash_embeds_weight):
    vocab, embed = hash_embeds_weight.shape
    (batch,) = user.shape
    idx = jnp.stack([user, item], axis=1).reshape(-1)
    tbl128 = jnp.pad(hash_embeds_weight, ((0, 0), (0, 128 - embed)))
    lookup = _make_lookup(vocab, 2 * batch)
    g = lookup(idx, tbl128)
    return g[:, :embed].reshape(batch, 2 * embed)
